# baseline (speedup 1.0000x reference)
import functools

import jax
import jax.numpy as jnp
from jax import lax
from jax.experimental import pallas as pl
from jax.experimental.pallas import tpu as pltpu
from jax.experimental.pallas import tpu_sc as plsc

D = 64
B_ = 4096
L_ = 200

NC = 2
NS = 16
NW = NC * NS

TOTAL = B_ * L_
PER_W = TOTAL // NW
CHUNK = 512
SUB = 128
NSUB = CHUNK // SUB
NCHUNK = PER_W // CHUNK

SCALE = 8.0


def _make_kernel():
  mesh = plsc.VectorSubcoreMesh(core_axis_name="c", subcore_axis_name="s")

  @functools.partial(
      pl.kernel,
      mesh=mesh,
      out_type=jax.ShapeDtypeStruct((TOTAL, D), jnp.float32),
      compiler_params=pltpu.CompilerParams(use_tc_tiling_on_sc=False),
      scratch_types=[
          pltpu.VMEM((NSUB, SUB), jnp.int32),
          pltpu.VMEM((CHUNK, D), jnp.float32),
          pltpu.SemaphoreType.DMA,
      ],
  )
  def k(table_hbm, idx_hbm, out_hbm, idx_v, rows_v, sem):
    wid = lax.axis_index("s") * NC + lax.axis_index("c")
    row0 = wid * (PER_W // SUB)

    def chunk_body(g, _):
      pltpu.sync_copy(idx_hbm.at[pl.ds(row0 + g * NSUB, NSUB)], idx_v)
      for s in range(NSUB):
        pltpu.async_copy(
            table_hbm.at[idx_v.at[s]],
            rows_v.at[pl.ds(s * SUB, SUB)],
            sem,
        )
      for s in range(NSUB):
        pltpu.make_async_copy(
            table_hbm.at[idx_v.at[s]],
            rows_v.at[pl.ds(s * SUB, SUB)],
            sem,
        ).wait()

      def scale_row(i, _):
        for j in range(D // 16):
          sl = pl.ds(j * 16, 16)
          rows_v[i, sl] = rows_v[i, sl] * SCALE
        return 0

      lax.fori_loop(0, CHUNK, scale_row, 0, unroll=2)

      pltpu.sync_copy(
          rows_v,
          out_hbm.at[pl.ds(wid * PER_W + g * CHUNK, CHUNK)],
      )
      return 0

    lax.fori_loop(0, NCHUNK, chunk_body, 0)

  return k


_kernel = _make_kernel()


def kernel(x, embedding):
  idx = x.reshape(TOTAL // SUB, SUB).astype(jnp.int32)
  out = _kernel(embedding, idx)
  return out.reshape(B_, L_, D)


# speedup vs baseline: 1.0892x; 1.0892x over previous
import functools

import jax
import jax.numpy as jnp
from jax import lax
from jax.experimental import pallas as pl
from jax.experimental.pallas import tpu as pltpu
from jax.experimental.pallas import tpu_sc as plsc

D = 64
B_ = 4096
L_ = 200

NC = 2
NS = 16
NW = NC * NS

TOTAL = B_ * L_
PER_W = TOTAL // NW
SUB = 128
IDX_ROWS = PER_W // SUB
HALF = 640
NGATHER = HALF // SUB
NHALF = PER_W // HALF

SCALE = 8.0


def _make_kernel():
  mesh = plsc.VectorSubcoreMesh(core_axis_name="c", subcore_axis_name="s")

  @functools.partial(
      pl.kernel,
      mesh=mesh,
      out_type=jax.ShapeDtypeStruct((TOTAL, D), jnp.float32),
      compiler_params=pltpu.CompilerParams(use_tc_tiling_on_sc=False),
      scratch_types=[
          pltpu.VMEM((IDX_ROWS, SUB), jnp.int32),
          pltpu.VMEM((2 * HALF, D), jnp.float32),
          pltpu.SemaphoreType.DMA,
          pltpu.SemaphoreType.DMA,
      ],
  )
  def k(table_hbm, idx_hbm, out_hbm, idx_v, rows_v, gsem, ssem):
    wid = lax.axis_index("s") * NC + lax.axis_index("c")
    row0 = wid * IDX_ROWS
    out0 = wid * PER_W

    pltpu.sync_copy(idx_hbm.at[pl.ds(row0, IDX_ROWS)], idx_v)

    def fire_gathers(h, buf):
      for s in range(NGATHER):
        pltpu.async_copy(
            table_hbm.at[idx_v.at[h * NGATHER + s]],
            rows_v.at[pl.ds(buf * HALF + s * SUB, SUB)],
            gsem,
        )

    def wait_gathers(h, buf):
      for s in range(NGATHER):
        pltpu.make_async_copy(
            table_hbm.at[idx_v.at[h * NGATHER + s]],
            rows_v.at[pl.ds(buf * HALF + s * SUB, SUB)],
            gsem,
        ).wait()

    def scale(buf):
      base = buf * HALF

      def scale_row(i, _):
        for j in range(D // 16):
          sl = pl.ds(j * 16, 16)
          rows_v[base + i, sl] = rows_v[base + i, sl] * SCALE
        return 0

      lax.fori_loop(0, HALF, scale_row, 0, unroll=4)

    def store_copy(h, buf):
      return pltpu.make_async_copy(
          rows_v.at[pl.ds(buf * HALF, HALF)],
          out_hbm.at[pl.ds(out0 + h * HALF, HALF)],
          ssem,
      )

    fire_gathers(0, 0)
    fire_gathers(1, 1)
    wait_gathers(0, 0)
    scale(0)
    store_copy(0, 0).start()

    def body(h, _):
      buf = lax.rem(h, 2)
      nxt = 1 - buf
      store_copy(h - 1, nxt).wait()
      fire_gathers(h + 1, nxt)
      wait_gathers(h, buf)
      scale(buf)
      store_copy(h, buf).start()
      return 0

    lax.fori_loop(1, NHALF - 1, body, 0)

    h = NHALF - 1
    buf = h % 2
    wait_gathers(h, buf)
    scale(buf)
    store_copy(h, buf).start()
    store_copy(h - 1, 1 - buf).wait()
    store_copy(h, buf).wait()

  return k


_kernel = _make_kernel()


def kernel(x, embedding):
  idx = x.reshape(TOTAL // SUB, SUB).astype(jnp.int32)
  out = _kernel(embedding, idx)
  return out.reshape(B_, L_, D)


# speedup vs baseline: 1.3286x; 1.2198x over previous
import functools

import jax
import jax.numpy as jnp
from jax import lax
from jax.experimental import pallas as pl
from jax.experimental.pallas import tpu as pltpu
from jax.experimental.pallas import tpu_sc as plsc

D = 64
DP = 128
B_ = 4096
L_ = 200

NC = 2
NS = 16
NW = NC * NS

TOTAL = B_ * L_
PER_W = TOTAL // NW
SUB = 128
IDX_ROWS = PER_W // SUB
HALF = 256
NGATHER = HALF // SUB
NHALF = PER_W // HALF

SCALE = 8.0


def _make_kernel():
  mesh = plsc.VectorSubcoreMesh(core_axis_name="c", subcore_axis_name="s")

  @functools.partial(
      pl.kernel,
      mesh=mesh,
      out_type=jax.ShapeDtypeStruct((TOTAL, DP), jnp.float32),
      compiler_params=pltpu.CompilerParams(use_tc_tiling_on_sc=True),
      scratch_types=[
          pltpu.VMEM((IDX_ROWS, SUB), jnp.int32),
          pltpu.VMEM((2 * HALF, DP), jnp.float32),
          pltpu.SemaphoreType.DMA,
          pltpu.SemaphoreType.DMA,
      ],
  )
  def k(table_hbm, idx_hbm, out_hbm, idx_v, rows_v, gsem, ssem):
    wid = lax.axis_index("s") * NC + lax.axis_index("c")
    row0 = wid * IDX_ROWS
    out0 = wid * PER_W

    pltpu.sync_copy(idx_hbm.at[pl.ds(row0, IDX_ROWS)], idx_v)

    def fire_gathers(h, buf):
      for s in range(NGATHER):
        pltpu.async_copy(
            table_hbm.at[idx_v.at[h * NGATHER + s]],
            rows_v.at[pl.ds(buf * HALF + s * SUB, SUB)],
            gsem,
        )

    def wait_gathers(h, buf):
      for s in range(NGATHER):
        pltpu.make_async_copy(
            table_hbm.at[idx_v.at[h * NGATHER + s]],
            rows_v.at[pl.ds(buf * HALF + s * SUB, SUB)],
            gsem,
        ).wait()

    def scale(buf):
      base = buf * HALF

      def scale_row(i, _):
        for j in range(D // 16):
          sl = pl.ds(j * 16, 16)
          rows_v[base + i, sl] = rows_v[base + i, sl] * SCALE
        return 0

      lax.fori_loop(0, HALF, scale_row, 0, unroll=4)

    def store_copy(h, buf):
      return pltpu.make_async_copy(
          rows_v.at[pl.ds(buf * HALF, HALF)],
          out_hbm.at[pl.ds(out0 + h * HALF, HALF)],
          ssem,
      )

    fire_gathers(0, 0)
    fire_gathers(1, 1)
    wait_gathers(0, 0)
    scale(0)
    store_copy(0, 0).start()

    def body(h, _):
      buf = lax.rem(h, 2)
      nxt = 1 - buf
      store_copy(h - 1, nxt).wait()
      fire_gathers(h + 1, nxt)
      wait_gathers(h, buf)
      scale(buf)
      store_copy(h, buf).start()
      return 0

    lax.fori_loop(1, NHALF - 1, body, 0)

    h = NHALF - 1
    buf = h % 2
    wait_gathers(h, buf)
    scale(buf)
    store_copy(h, buf).start()
    store_copy(h - 1, 1 - buf).wait()
    store_copy(h, buf).wait()

  return k


_kernel = _make_kernel()


def kernel(x, embedding):
  table = jnp.pad(embedding, ((0, 0), (0, DP - D)))
  idx = x.reshape(TOTAL // SUB, SUB).astype(jnp.int32)
  out = _kernel(table, idx)
  return out.reshape(B_, L_, DP)[:, :, :D]
